# baseline (speedup 1.0000x reference)
import functools

import jax
import jax.numpy as jnp
from jax import lax
from jax.experimental import pallas as pl
from jax.experimental.pallas import tpu as pltpu
from jax.experimental.pallas import tpu_sc as plsc

N = 10000
E = 160000
D = 256
H = 256
NC = 2
NS = 16
NPAD = 10240
STRIPE = NPAD // NS
CH = 125

_mesh = plsc.VectorSubcoreMesh(
    core_axis_name="c", subcore_axis_name="s", num_cores=NC, num_subcores=NS
)

def _deg_body(dst_hbm, ones_hbm, zeros_hbm, out_hbm, idx_v, ones_v, zer_v, acc_s):
    cid = lax.axis_index("c")
    sid = lax.axis_index("s")
    pltpu.sync_copy(dst_hbm.at[cid, sid], idx_v)
    pltpu.sync_copy(ones_hbm, ones_v)
    pltpu.sync_copy(zeros_hbm, zer_v)
    pltpu.sync_copy(zer_v, acc_s.at[pl.ds(sid * STRIPE, STRIPE)])
    plsc.subcore_barrier()

    def body(j, c):
        pltpu.sync_copy(ones_v, acc_s.at[idx_v.at[j]], add=True)
        return c

    lax.fori_loop(0, E // (NC * NS * CH), body, 0)
    plsc.subcore_barrier()
    pltpu.sync_copy(
        acc_s.at[pl.ds(sid * STRIPE, STRIPE)],
        out_hbm.at[cid, pl.ds(sid * STRIPE, STRIPE)],
    )


_deg = pl.kernel(
    _deg_body,
    out_type=jax.ShapeDtypeStruct((NC, NPAD), jnp.float32),
    mesh=_mesh,
    scratch_types=[
        pltpu.VMEM((E // (NC * NS * CH), CH), jnp.int32),
        pltpu.VMEM((CH,), jnp.float32),
        pltpu.VMEM((STRIPE,), jnp.float32),
        pltpu.VMEM_SHARED((NPAD,), jnp.float32),
    ],
)

FH = 128
NB = 2
G = 20


def _edge_ring_blk(hs_hbm, acc_s, src_hbm, dst_hbm, sid, srcv, dstv, rows,
                   sems, nch):
    gsems, ssems = sems[:NB], sems[NB:]

    def blk(bi, c):
        pltpu.sync_copy(src_hbm.at[sid, pl.ds(bi * G, G)], srcv)
        pltpu.sync_copy(dst_hbm.at[sid, pl.ds(bi * G, G)], dstv)
        for b in range(NB):
            pltpu.async_copy(hs_hbm.at[srcv.at[b]], rows.at[b], gsems[b])

        def group(g, c2):
            j0 = g * NB
            for b in range(NB):
                jj = j0 + b
                pltpu.make_async_copy(hs_hbm.at[srcv.at[jj]], rows.at[b],
                                      gsems[b]).wait()
                pltpu.async_copy(rows.at[b], acc_s.at[dstv.at[jj]], ssems[b],
                                 add=True)
            for b in range(NB):
                jj = j0 + b
                pltpu.make_async_copy(rows.at[b], acc_s.at[dstv.at[jj]],
                                      ssems[b]).wait()

                @pl.when(jj + NB < G)
                def _():
                    pltpu.async_copy(hs_hbm.at[srcv.at[jj + NB]], rows.at[b],
                                     gsems[b])

            return c2

        lax.fori_loop(0, G // NB, group, 0)
        return c

    lax.fori_loop(0, nch // G, blk, 0)


def _edge_ring(hs_hbm, acc_s, srcv, dstv, rows, sems, nch):
    gsems, ssems = sems[:NB], sems[NB:]
    for b in range(NB):
        pltpu.async_copy(hs_hbm.at[srcv.at[b]], rows.at[b], gsems[b])

    def group(g, c):
        j0 = g * NB
        for b in range(NB):
            jj = j0 + b
            pltpu.make_async_copy(hs_hbm.at[srcv.at[jj]], rows.at[b], gsems[b]).wait()
            pltpu.async_copy(rows.at[b], acc_s.at[dstv.at[jj]], ssems[b], add=True)
        for b in range(NB):
            jj = j0 + b
            pltpu.make_async_copy(rows.at[b], acc_s.at[dstv.at[jj]], ssems[b]).wait()

            @pl.when(jj + NB < nch)
            def _():
                pltpu.async_copy(hs_hbm.at[srcv.at[jj + NB]], rows.at[b], gsems[b])

        return c

    lax.fori_loop(0, nch // NB, group, 0)


def _agg1_body(hsA, hsB, src16, dst16, zeros_hbm, out_hbm,
               srcv, dstv, rows, zer, acc_s, *sems):
    cid = lax.axis_index("c")
    sid = lax.axis_index("s")
    nch = E // (NS * CH)
    pltpu.sync_copy(zeros_hbm, zer)
    for kk in range(STRIPE // 8):
        pltpu.sync_copy(zer, acc_s.at[pl.ds(sid * STRIPE + kk * 8, 8)])
    plsc.subcore_barrier()

    @pl.when(cid == 0)
    def _():
        _edge_ring_blk(hsA, acc_s, src16, dst16, sid, srcv, dstv, rows, sems, nch)

    @pl.when(cid == 1)
    def _():
        _edge_ring_blk(hsB, acc_s, src16, dst16, sid, srcv, dstv, rows, sems, nch)

    plsc.subcore_barrier()
    pltpu.sync_copy(
        acc_s.at[pl.ds(sid * STRIPE, STRIPE)],
        out_hbm.at[cid, pl.ds(sid * STRIPE, STRIPE)],
    )


_agg1 = pl.kernel(
    _agg1_body,
    out_type=jax.ShapeDtypeStruct((NC, NPAD, FH), jnp.float32),
    mesh=_mesh,
    scratch_types=[
        pltpu.VMEM((G, CH), jnp.int32),
        pltpu.VMEM((G, CH), jnp.int32),
        pltpu.VMEM((NB, CH, FH), jnp.float32),
        pltpu.VMEM((8, FH), jnp.float32),
        pltpu.VMEM_SHARED((NPAD, FH), jnp.float32),
    ] + [pltpu.SemaphoreType.DMA] * (2 * NB),
    compiler_params=pltpu.CompilerParams(use_tc_tiling_on_sc=False),
)

def _agg2_body(gs_hbm, src4, dst4, zeros_hbm, out_hbm, srcv, dstv, rows, zer,
               acc_s, *sems):
    cid = lax.axis_index("c")
    sid = lax.axis_index("s")
    nch = E // (NC * NS * CH)
    pltpu.sync_copy(src4.at[cid, sid], srcv)
    pltpu.sync_copy(dst4.at[cid, sid], dstv)
    pltpu.sync_copy(zeros_hbm, zer)
    pltpu.sync_copy(zer, acc_s.at[pl.ds(sid * STRIPE, STRIPE)])
    plsc.subcore_barrier()
    _edge_ring(gs_hbm, acc_s, srcv, dstv, rows, sems, nch)
    plsc.subcore_barrier()
    pltpu.sync_copy(
        acc_s.at[pl.ds(sid * STRIPE, STRIPE)],
        out_hbm.at[cid, pl.ds(sid * STRIPE, STRIPE)],
    )


_agg2 = pl.kernel(
    _agg2_body,
    out_type=jax.ShapeDtypeStruct((NC, NPAD, 16), jnp.float32),
    mesh=_mesh,
    scratch_types=[
        pltpu.VMEM((E // (NC * NS * CH), CH), jnp.int32),
        pltpu.VMEM((E // (NC * NS * CH), CH), jnp.int32),
        pltpu.VMEM((NB, CH, 16), jnp.float32),
        pltpu.VMEM((STRIPE, 16), jnp.float32),
        pltpu.VMEM_SHARED((NPAD, 16), jnp.float32),
    ] + [pltpu.SemaphoreType.DMA] * (2 * NB),
    compiler_params=pltpu.CompilerParams(use_tc_tiling_on_sc=False),
)

BM = 1024


def _k2_body(x_ref, w1_ref, degp_ref, hsl_ref, hsh_ref, dinv_ref):
    i = pl.program_id(0)
    deg = degp_ref[0, pl.ds(i * BM, BM)] + degp_ref[1, pl.ds(i * BM, BM)] + 1.0
    dinv = lax.rsqrt(deg)
    h = jnp.dot(x_ref[...], w1_ref[...], preferred_element_type=jnp.float32)
    hs = h * dinv[:, None]
    hsl_ref[...] = hs[:, :FH]
    hsh_ref[...] = hs[:, FH:]
    dinv_ref[pl.ds(i * BM, BM)] = dinv


def _k2(x, W1, degp):
    return pl.pallas_call(
        _k2_body,
        grid=(pl.cdiv(N, BM),),
        in_specs=[
            pl.BlockSpec((BM, D), lambda i: (i, 0)),
            pl.BlockSpec((D, H), lambda i: (0, 0)),
            pl.BlockSpec((NC, NPAD), lambda i: (0, 0)),
        ],
        out_specs=[
            pl.BlockSpec((BM, FH), lambda i: (i, 0)),
            pl.BlockSpec((BM, FH), lambda i: (i, 0)),
            pl.BlockSpec((NPAD,), lambda i: (0,)),
        ],
        out_shape=[
            jax.ShapeDtypeStruct((N, FH), jnp.float32),
            jax.ShapeDtypeStruct((N, FH), jnp.float32),
            jax.ShapeDtypeStruct((NPAD,), jnp.float32),
        ],
    )(x, W1, degp)


def _k4_body(t_ref, hsl_ref, hsh_ref, dinv_ref, b1_ref, w2_ref, gs_ref):
    i = pl.program_id(0)
    dinv = dinv_ref[pl.ds(i * BM, BM)]
    b1 = b1_ref[...]
    al = (t_ref[0] + hsl_ref[...]) * dinv[:, None] + b1[None, :FH]
    ah = (t_ref[1] + hsh_ref[...]) * dinv[:, None] + b1[None, FH:]
    g = (jnp.dot(jnp.maximum(al, 0.0), w2_ref[pl.ds(0, FH), :],
                 preferred_element_type=jnp.float32)
         + jnp.dot(jnp.maximum(ah, 0.0), w2_ref[pl.ds(FH, FH), :],
                   preferred_element_type=jnp.float32))
    gs_ref[...] = g * dinv[:, None]


def _k4(t, hsl, hsh, dinv, b1, W2p):
    return pl.pallas_call(
        _k4_body,
        grid=(pl.cdiv(N, BM),),
        in_specs=[
            pl.BlockSpec((NC, BM, FH), lambda i: (0, i, 0)),
            pl.BlockSpec((BM, FH), lambda i: (i, 0)),
            pl.BlockSpec((BM, FH), lambda i: (i, 0)),
            pl.BlockSpec((NPAD,), lambda i: (0,)),
            pl.BlockSpec((H,), lambda i: (0,)),
            pl.BlockSpec((H, 16), lambda i: (0, 0)),
        ],
        out_specs=pl.BlockSpec((BM, 16), lambda i: (i, 0)),
        out_shape=jax.ShapeDtypeStruct((N, 16), jnp.float32),
    )(t, hsl, hsh, dinv, b1, W2p)


def _k6_body(t2a_ref, t2b_ref, gs_ref, dinv_ref, b2_ref, out_ref):
    i = pl.program_id(0)
    dinv = dinv_ref[pl.ds(i * BM, BM)]
    z = (t2a_ref[...] + t2b_ref[...] + gs_ref[...]) * dinv[:, None]
    z2 = z[:, :2] + b2_ref[...][None, :]
    m = jnp.max(z2, axis=1, keepdims=True)
    lse = m + jnp.log(jnp.sum(jnp.exp(z2 - m), axis=1, keepdims=True))
    out_ref[...] = z2 - lse


def _k6(t2a, t2b, gs, dinv, b2):
    return pl.pallas_call(
        _k6_body,
        grid=(pl.cdiv(N, BM),),
        in_specs=[
            pl.BlockSpec((BM, 16), lambda i: (i, 0)),
            pl.BlockSpec((BM, 16), lambda i: (i, 0)),
            pl.BlockSpec((BM, 16), lambda i: (i, 0)),
            pl.BlockSpec((NPAD,), lambda i: (0,)),
            pl.BlockSpec((2,), lambda i: (0,)),
        ],
        out_specs=pl.BlockSpec((BM, 2), lambda i: (i, 0)),
        out_shape=jax.ShapeDtypeStruct((N, 2), jnp.float32),
    )(t2a, t2b, gs, dinv, b2)


def kernel(x, edge_index, W1, b1, W2, b2):
    src = edge_index[0]
    dst = edge_index[1]
    src16 = src.reshape(NS, E // (NS * CH), CH)
    dst16 = dst.reshape(NS, E // (NS * CH), CH)
    src4 = src.reshape(NC, NS, E // (NC * NS * CH), CH)
    dst4 = dst.reshape(NC, NS, E // (NC * NS * CH), CH)

    ones_ch = jnp.ones((CH,), jnp.float32)
    zer_stripe = jnp.zeros((STRIPE,), jnp.float32)
    zer_128 = jnp.zeros((8, FH), jnp.float32)
    zer_s16 = jnp.zeros((STRIPE, 16), jnp.float32)
    W2p = jnp.zeros((H, 16), jnp.float32).at[:, :2].set(W2)

    degp = _deg(dst4, ones_ch, zer_stripe)
    hsl, hsh, dinv = _k2(x, W1, degp)
    t = _agg1(hsl, hsh, src16, dst16, zer_128)
    gs = _k4(t, hsl, hsh, dinv, b1, W2p)
    t2 = _agg2(gs, src4, dst4, zer_s16)
    return _k6(t2[0], t2[1], gs, dinv, b2)


# speedup vs baseline: 1.0953x; 1.0953x over previous
import functools

import jax
import jax.numpy as jnp
from jax import lax
from jax.experimental import pallas as pl
from jax.experimental.pallas import tpu as pltpu
from jax.experimental.pallas import tpu_sc as plsc

N = 10000
E = 160000
D = 256
H = 256
NC = 2
NS = 16
NPAD = 10240
STRIPE = NPAD // NS
CH = 125

_mesh = plsc.VectorSubcoreMesh(
    core_axis_name="c", subcore_axis_name="s", num_cores=NC, num_subcores=NS
)

def _deg_body(dst_hbm, ones_hbm, zeros_hbm, out_hbm, idx_v, ones_v, zer_v, acc_s):
    cid = lax.axis_index("c")
    sid = lax.axis_index("s")
    pltpu.sync_copy(dst_hbm.at[cid, sid], idx_v)
    pltpu.sync_copy(ones_hbm, ones_v)
    pltpu.sync_copy(zeros_hbm, zer_v)
    pltpu.sync_copy(zer_v, acc_s.at[pl.ds(sid * STRIPE, STRIPE)])
    plsc.subcore_barrier()

    def body(j, c):
        pltpu.sync_copy(ones_v, acc_s.at[idx_v.at[j]], add=True)
        return c

    lax.fori_loop(0, E // (NC * NS * CH), body, 0)
    plsc.subcore_barrier()
    pltpu.sync_copy(
        acc_s.at[pl.ds(sid * STRIPE, STRIPE)],
        out_hbm.at[cid, pl.ds(sid * STRIPE, STRIPE)],
    )


_deg = pl.kernel(
    _deg_body,
    out_type=jax.ShapeDtypeStruct((NC, NPAD), jnp.float32),
    mesh=_mesh,
    scratch_types=[
        pltpu.VMEM((E // (NC * NS * CH), CH), jnp.int32),
        pltpu.VMEM((CH,), jnp.float32),
        pltpu.VMEM((STRIPE,), jnp.float32),
        pltpu.VMEM_SHARED((NPAD,), jnp.float32),
    ],
)

FH = 128
NB = 4
NB1 = 4
CH1 = 50
G = 20


def _edge_ring_blk(hs_hbm, acc_s, src_hbm, dst_hbm, sid, srcv, dstv, rows,
                   sems, nch):
    gsems, ssems = sems[:NB1], sems[NB1:]

    def blk(bi, c):
        pltpu.sync_copy(src_hbm.at[sid, pl.ds(bi * G, G)], srcv)
        pltpu.sync_copy(dst_hbm.at[sid, pl.ds(bi * G, G)], dstv)
        for b in range(NB1):
            pltpu.async_copy(hs_hbm.at[srcv.at[b]], rows.at[b], gsems[b])

        def group(g, c2):
            j0 = g * NB1
            for b in range(NB1):
                jj = j0 + b
                pltpu.make_async_copy(hs_hbm.at[srcv.at[jj]], rows.at[b],
                                      gsems[b]).wait()
                pltpu.async_copy(rows.at[b], acc_s.at[dstv.at[jj]], ssems[b],
                                 add=True)
            for b in range(NB1):
                jj = j0 + b
                pltpu.make_async_copy(rows.at[b], acc_s.at[dstv.at[jj]],
                                      ssems[b]).wait()

                @pl.when(jj + NB1 < G)
                def _():
                    pltpu.async_copy(hs_hbm.at[srcv.at[jj + NB1]], rows.at[b],
                                     gsems[b])

            return c2

        lax.fori_loop(0, G // NB1, group, 0)
        return c

    lax.fori_loop(0, nch // G, blk, 0)


def _edge_ring(hs_hbm, acc_s, srcv, dstv, rows, sems, nch):
    gsems, ssems = sems[:NB], sems[NB:]
    for b in range(NB):
        pltpu.async_copy(hs_hbm.at[srcv.at[b]], rows.at[b], gsems[b])

    def group(g, c):
        j0 = g * NB
        for b in range(NB):
            jj = j0 + b
            pltpu.make_async_copy(hs_hbm.at[srcv.at[jj]], rows.at[b], gsems[b]).wait()
            pltpu.async_copy(rows.at[b], acc_s.at[dstv.at[jj]], ssems[b], add=True)
        for b in range(NB):
            jj = j0 + b
            pltpu.make_async_copy(rows.at[b], acc_s.at[dstv.at[jj]], ssems[b]).wait()

            @pl.when(jj + NB < nch)
            def _():
                pltpu.async_copy(hs_hbm.at[srcv.at[jj + NB]], rows.at[b], gsems[b])

        return c

    lax.fori_loop(0, nch // NB, group, 0)


def _agg1_body(hsA, hsB, src16, dst16, zeros_hbm, out_hbm,
               srcv, dstv, rows, zer, acc_s, *sems):
    cid = lax.axis_index("c")
    sid = lax.axis_index("s")
    nch = E // (NS * CH1)
    pltpu.sync_copy(zeros_hbm, zer)
    for kk in range(STRIPE // 8):
        pltpu.sync_copy(zer, acc_s.at[pl.ds(sid * STRIPE + kk * 8, 8)])
    plsc.subcore_barrier()

    @pl.when(cid == 0)
    def _():
        _edge_ring_blk(hsA, acc_s, src16, dst16, sid, srcv, dstv, rows, sems, nch)

    @pl.when(cid == 1)
    def _():
        _edge_ring_blk(hsB, acc_s, src16, dst16, sid, srcv, dstv, rows, sems, nch)

    plsc.subcore_barrier()
    pltpu.sync_copy(
        acc_s.at[pl.ds(sid * STRIPE, STRIPE)],
        out_hbm.at[cid, pl.ds(sid * STRIPE, STRIPE)],
    )


_agg1 = pl.kernel(
    _agg1_body,
    out_type=jax.ShapeDtypeStruct((NC, NPAD, FH), jnp.float32),
    mesh=_mesh,
    scratch_types=[
        pltpu.VMEM((G, CH1), jnp.int32),
        pltpu.VMEM((G, CH1), jnp.int32),
        pltpu.VMEM((NB1, CH1, FH), jnp.float32),
        pltpu.VMEM((8, FH), jnp.float32),
        pltpu.VMEM_SHARED((NPAD, FH), jnp.float32),
    ] + [pltpu.SemaphoreType.DMA] * (2 * NB1),
    compiler_params=pltpu.CompilerParams(use_tc_tiling_on_sc=False),
)

def _agg2_body(gs_hbm, src4, dst4, zeros_hbm, out_hbm, srcv, dstv, rows, zer,
               acc_s, *sems):
    cid = lax.axis_index("c")
    sid = lax.axis_index("s")
    nch = E // (NC * NS * CH)
    pltpu.sync_copy(src4.at[cid, sid], srcv)
    pltpu.sync_copy(dst4.at[cid, sid], dstv)
    pltpu.sync_copy(zeros_hbm, zer)
    pltpu.sync_copy(zer, acc_s.at[pl.ds(sid * STRIPE, STRIPE)])
    plsc.subcore_barrier()
    _edge_ring(gs_hbm, acc_s, srcv, dstv, rows, sems, nch)
    plsc.subcore_barrier()
    pltpu.sync_copy(
        acc_s.at[pl.ds(sid * STRIPE, STRIPE)],
        out_hbm.at[cid, pl.ds(sid * STRIPE, STRIPE)],
    )


_agg2 = pl.kernel(
    _agg2_body,
    out_type=jax.ShapeDtypeStruct((NC, NPAD, 16), jnp.float32),
    mesh=_mesh,
    scratch_types=[
        pltpu.VMEM((E // (NC * NS * CH), CH), jnp.int32),
        pltpu.VMEM((E // (NC * NS * CH), CH), jnp.int32),
        pltpu.VMEM((NB, CH, 16), jnp.float32),
        pltpu.VMEM((STRIPE, 16), jnp.float32),
        pltpu.VMEM_SHARED((NPAD, 16), jnp.float32),
    ] + [pltpu.SemaphoreType.DMA] * (2 * NB),
    compiler_params=pltpu.CompilerParams(use_tc_tiling_on_sc=False),
)

BM = 1024


def _k2_body(x_ref, w1_ref, degp_ref, hsl_ref, hsh_ref, dinv_ref):
    i = pl.program_id(0)
    deg = degp_ref[0, pl.ds(i * BM, BM)] + degp_ref[1, pl.ds(i * BM, BM)] + 1.0
    dinv = lax.rsqrt(deg)
    h = jnp.dot(x_ref[...], w1_ref[...], preferred_element_type=jnp.float32)
    hs = h * dinv[:, None]
    hsl_ref[...] = hs[:, :FH]
    hsh_ref[...] = hs[:, FH:]
    dinv_ref[pl.ds(i * BM, BM)] = dinv


def _k2(x, W1, degp):
    return pl.pallas_call(
        _k2_body,
        grid=(pl.cdiv(N, BM),),
        in_specs=[
            pl.BlockSpec((BM, D), lambda i: (i, 0)),
            pl.BlockSpec((D, H), lambda i: (0, 0)),
            pl.BlockSpec((NC, NPAD), lambda i: (0, 0)),
        ],
        out_specs=[
            pl.BlockSpec((BM, FH), lambda i: (i, 0)),
            pl.BlockSpec((BM, FH), lambda i: (i, 0)),
            pl.BlockSpec((NPAD,), lambda i: (0,)),
        ],
        out_shape=[
            jax.ShapeDtypeStruct((N, FH), jnp.float32),
            jax.ShapeDtypeStruct((N, FH), jnp.float32),
            jax.ShapeDtypeStruct((NPAD,), jnp.float32),
        ],
    )(x, W1, degp)


def _k4_body(t_ref, hsl_ref, hsh_ref, dinv_ref, b1_ref, w2_ref, gs_ref):
    i = pl.program_id(0)
    dinv = dinv_ref[pl.ds(i * BM, BM)]
    b1 = b1_ref[...]
    al = (t_ref[0] + hsl_ref[...]) * dinv[:, None] + b1[None, :FH]
    ah = (t_ref[1] + hsh_ref[...]) * dinv[:, None] + b1[None, FH:]
    g = (jnp.dot(jnp.maximum(al, 0.0), w2_ref[pl.ds(0, FH), :],
                 preferred_element_type=jnp.float32)
         + jnp.dot(jnp.maximum(ah, 0.0), w2_ref[pl.ds(FH, FH), :],
                   preferred_element_type=jnp.float32))
    gs_ref[...] = g * dinv[:, None]


def _k4(t, hsl, hsh, dinv, b1, W2p):
    return pl.pallas_call(
        _k4_body,
        grid=(pl.cdiv(N, BM),),
        in_specs=[
            pl.BlockSpec((NC, BM, FH), lambda i: (0, i, 0)),
            pl.BlockSpec((BM, FH), lambda i: (i, 0)),
            pl.BlockSpec((BM, FH), lambda i: (i, 0)),
            pl.BlockSpec((NPAD,), lambda i: (0,)),
            pl.BlockSpec((H,), lambda i: (0,)),
            pl.BlockSpec((H, 16), lambda i: (0, 0)),
        ],
        out_specs=pl.BlockSpec((BM, 16), lambda i: (i, 0)),
        out_shape=jax.ShapeDtypeStruct((N, 16), jnp.float32),
    )(t, hsl, hsh, dinv, b1, W2p)


def _k6_body(t2a_ref, t2b_ref, gs_ref, dinv_ref, b2_ref, out_ref):
    i = pl.program_id(0)
    dinv = dinv_ref[pl.ds(i * BM, BM)]
    z = (t2a_ref[...] + t2b_ref[...] + gs_ref[...]) * dinv[:, None]
    z2 = z[:, :2] + b2_ref[...][None, :]
    m = jnp.max(z2, axis=1, keepdims=True)
    lse = m + jnp.log(jnp.sum(jnp.exp(z2 - m), axis=1, keepdims=True))
    out_ref[...] = z2 - lse


def _k6(t2a, t2b, gs, dinv, b2):
    return pl.pallas_call(
        _k6_body,
        grid=(pl.cdiv(N, BM),),
        in_specs=[
            pl.BlockSpec((BM, 16), lambda i: (i, 0)),
            pl.BlockSpec((BM, 16), lambda i: (i, 0)),
            pl.BlockSpec((BM, 16), lambda i: (i, 0)),
            pl.BlockSpec((NPAD,), lambda i: (0,)),
            pl.BlockSpec((2,), lambda i: (0,)),
        ],
        out_specs=pl.BlockSpec((BM, 2), lambda i: (i, 0)),
        out_shape=jax.ShapeDtypeStruct((N, 2), jnp.float32),
    )(t2a, t2b, gs, dinv, b2)


def kernel(x, edge_index, W1, b1, W2, b2):
    src = edge_index[0]
    dst = edge_index[1]
    src16 = src.reshape(NS, E // (NS * CH1), CH1)
    dst16 = dst.reshape(NS, E // (NS * CH1), CH1)
    src4 = src.reshape(NC, NS, E // (NC * NS * CH), CH)
    dst4 = dst.reshape(NC, NS, E // (NC * NS * CH), CH)

    ones_ch = jnp.ones((CH,), jnp.float32)
    zer_stripe = jnp.zeros((STRIPE,), jnp.float32)
    zer_128 = jnp.zeros((8, FH), jnp.float32)
    zer_s16 = jnp.zeros((STRIPE, 16), jnp.float32)
    W2p = jnp.zeros((H, 16), jnp.float32).at[:, :2].set(W2)

    degp = _deg(dst4, ones_ch, zer_stripe)
    hsl, hsh, dinv = _k2(x, W1, degp)
    t = _agg1(hsl, hsh, src16, dst16, zer_128)
    gs = _k4(t, hsl, hsh, dinv, b1, W2p)
    t2 = _agg2(gs, src4, dst4, zer_s16)
    return _k6(t2[0], t2[1], gs, dinv, b2)


# speedup vs baseline: 1.1053x; 1.0091x over previous
import functools

import jax
import jax.numpy as jnp
from jax import lax
from jax.experimental import pallas as pl
from jax.experimental.pallas import tpu as pltpu
from jax.experimental.pallas import tpu_sc as plsc

N = 10000
E = 160000
D = 256
H = 256
NC = 2
NS = 16
NPAD = 10240
STRIPE = NPAD // NS
CH = 125

_mesh = plsc.VectorSubcoreMesh(
    core_axis_name="c", subcore_axis_name="s", num_cores=NC, num_subcores=NS
)

def _deg_body(dst_hbm, ones_hbm, zeros_hbm, out_hbm, idx_v, ones_v, zer_v, acc_s):
    cid = lax.axis_index("c")
    sid = lax.axis_index("s")
    pltpu.sync_copy(dst_hbm.at[cid, sid], idx_v)
    pltpu.sync_copy(ones_hbm, ones_v)
    pltpu.sync_copy(zeros_hbm, zer_v)
    pltpu.sync_copy(zer_v, acc_s.at[pl.ds(sid * STRIPE, STRIPE)])
    plsc.subcore_barrier()

    def body(j, c):
        pltpu.sync_copy(ones_v, acc_s.at[idx_v.at[j]], add=True)
        return c

    lax.fori_loop(0, E // (NC * NS * CH), body, 0)
    plsc.subcore_barrier()
    pltpu.sync_copy(
        acc_s.at[pl.ds(sid * STRIPE, STRIPE)],
        out_hbm.at[cid, pl.ds(sid * STRIPE, STRIPE)],
    )


_deg = pl.kernel(
    _deg_body,
    out_type=jax.ShapeDtypeStruct((NC, NPAD), jnp.float32),
    mesh=_mesh,
    scratch_types=[
        pltpu.VMEM((E // (NC * NS * CH), CH), jnp.int32),
        pltpu.VMEM((CH,), jnp.float32),
        pltpu.VMEM((STRIPE,), jnp.float32),
        pltpu.VMEM_SHARED((NPAD,), jnp.float32),
    ],
)

FH = 128
NB = 4
NB1 = 5
CH1 = 40
G = 25


def _edge_ring_blk(hs_hbm, acc_s, src_hbm, dst_hbm, sid, srcv, dstv, rows,
                   sems, nch):
    gsems, ssems = sems[:NB1], sems[NB1:]

    ept = E // NS

    def blk(bi, c):
        base = sid * ept + bi * G * CH1
        pltpu.sync_copy(src_hbm.at[pl.ds(base, G * CH1)], srcv)
        pltpu.sync_copy(dst_hbm.at[pl.ds(base, G * CH1)], dstv)
        for b in range(NB1):
            pltpu.async_copy(hs_hbm.at[srcv.at[pl.ds(b * CH1, CH1)]], rows.at[b],
                             gsems[b])

        def group(g, c2):
            j0 = g * NB1
            for b in range(NB1):
                jj = j0 + b
                pltpu.make_async_copy(
                    hs_hbm.at[srcv.at[pl.ds(jj * CH1, CH1)]], rows.at[b],
                    gsems[b]).wait()
                pltpu.async_copy(rows.at[b],
                                 acc_s.at[dstv.at[pl.ds(jj * CH1, CH1)]],
                                 ssems[b], add=True)
            for b in range(NB1):
                jj = j0 + b
                pltpu.make_async_copy(
                    rows.at[b], acc_s.at[dstv.at[pl.ds(jj * CH1, CH1)]],
                    ssems[b]).wait()

                @pl.when(jj + NB1 < G)
                def _():
                    pltpu.async_copy(
                        hs_hbm.at[srcv.at[pl.ds((jj + NB1) * CH1, CH1)]],
                        rows.at[b], gsems[b])

            return c2

        lax.fori_loop(0, G // NB1, group, 0)
        return c

    lax.fori_loop(0, nch // G, blk, 0)


def _edge_ring(hs_hbm, acc_s, srcv, dstv, rows, sems, nch):
    gsems, ssems = sems[:NB], sems[NB:]
    for b in range(NB):
        pltpu.async_copy(hs_hbm.at[srcv.at[b]], rows.at[b], gsems[b])

    def group(g, c):
        j0 = g * NB
        for b in range(NB):
            jj = j0 + b
            pltpu.make_async_copy(hs_hbm.at[srcv.at[jj]], rows.at[b], gsems[b]).wait()
            pltpu.async_copy(rows.at[b], acc_s.at[dstv.at[jj]], ssems[b], add=True)
        for b in range(NB):
            jj = j0 + b
            pltpu.make_async_copy(rows.at[b], acc_s.at[dstv.at[jj]], ssems[b]).wait()

            @pl.when(jj + NB < nch)
            def _():
                pltpu.async_copy(hs_hbm.at[srcv.at[jj + NB]], rows.at[b], gsems[b])

        return c

    lax.fori_loop(0, nch // NB, group, 0)


def _agg1_body(hsA, hsB, src16, dst16, zeros_hbm, out_hbm,
               srcv, dstv, rows, zer, acc_s, *sems):
    cid = lax.axis_index("c")
    sid = lax.axis_index("s")
    nch = E // (NS * CH1)
    pltpu.sync_copy(zeros_hbm, zer)
    for kk in range(STRIPE // 8):
        pltpu.sync_copy(zer, acc_s.at[pl.ds(sid * STRIPE + kk * 8, 8)])
    plsc.subcore_barrier()

    @pl.when(cid == 0)
    def _():
        _edge_ring_blk(hsA, acc_s, src16, dst16, sid, srcv, dstv, rows, sems, nch)

    @pl.when(cid == 1)
    def _():
        _edge_ring_blk(hsB, acc_s, src16, dst16, sid, srcv, dstv, rows, sems, nch)

    plsc.subcore_barrier()
    pltpu.sync_copy(
        acc_s.at[pl.ds(sid * STRIPE, STRIPE)],
        out_hbm.at[cid, pl.ds(sid * STRIPE, STRIPE)],
    )


_agg1 = pl.kernel(
    _agg1_body,
    out_type=jax.ShapeDtypeStruct((NC, NPAD, FH), jnp.float32),
    mesh=_mesh,
    scratch_types=[
        pltpu.VMEM((G * CH1,), jnp.int32),
        pltpu.VMEM((G * CH1,), jnp.int32),
        pltpu.VMEM((NB1, CH1, FH), jnp.float32),
        pltpu.VMEM((8, FH), jnp.float32),
        pltpu.VMEM_SHARED((NPAD, FH), jnp.float32),
    ] + [pltpu.SemaphoreType.DMA] * (2 * NB1),
    compiler_params=pltpu.CompilerParams(use_tc_tiling_on_sc=False),
)

def _agg2_body(gs_hbm, src4, dst4, zeros_hbm, out_hbm, srcv, dstv, rows, zer,
               acc_s, *sems):
    cid = lax.axis_index("c")
    sid = lax.axis_index("s")
    nch = E // (NC * NS * CH)
    pltpu.sync_copy(src4.at[cid, sid], srcv)
    pltpu.sync_copy(dst4.at[cid, sid], dstv)
    pltpu.sync_copy(zeros_hbm, zer)
    pltpu.sync_copy(zer, acc_s.at[pl.ds(sid * STRIPE, STRIPE)])
    plsc.subcore_barrier()
    _edge_ring(gs_hbm, acc_s, srcv, dstv, rows, sems, nch)
    plsc.subcore_barrier()
    pltpu.sync_copy(
        acc_s.at[pl.ds(sid * STRIPE, STRIPE)],
        out_hbm.at[cid, pl.ds(sid * STRIPE, STRIPE)],
    )


_agg2 = pl.kernel(
    _agg2_body,
    out_type=jax.ShapeDtypeStruct((NC, NPAD, 16), jnp.float32),
    mesh=_mesh,
    scratch_types=[
        pltpu.VMEM((E // (NC * NS * CH), CH), jnp.int32),
        pltpu.VMEM((E // (NC * NS * CH), CH), jnp.int32),
        pltpu.VMEM((NB, CH, 16), jnp.float32),
        pltpu.VMEM((STRIPE, 16), jnp.float32),
        pltpu.VMEM_SHARED((NPAD, 16), jnp.float32),
    ] + [pltpu.SemaphoreType.DMA] * (2 * NB),
    compiler_params=pltpu.CompilerParams(use_tc_tiling_on_sc=False),
)

BM = 1024


def _k2_body(x_ref, w1_ref, degp_ref, hsl_ref, hsh_ref, dinv_ref):
    i = pl.program_id(0)
    deg = degp_ref[0, pl.ds(i * BM, BM)] + degp_ref[1, pl.ds(i * BM, BM)] + 1.0
    dinv = lax.rsqrt(deg)
    h = jnp.dot(x_ref[...], w1_ref[...], preferred_element_type=jnp.float32)
    hs = h * dinv[:, None]
    hsl_ref[...] = hs[:, :FH]
    hsh_ref[...] = hs[:, FH:]
    dinv_ref[pl.ds(i * BM, BM)] = dinv


def _k2(x, W1, degp):
    return pl.pallas_call(
        _k2_body,
        grid=(pl.cdiv(N, BM),),
        in_specs=[
            pl.BlockSpec((BM, D), lambda i: (i, 0)),
            pl.BlockSpec((D, H), lambda i: (0, 0)),
            pl.BlockSpec((NC, NPAD), lambda i: (0, 0)),
        ],
        out_specs=[
            pl.BlockSpec((BM, FH), lambda i: (i, 0)),
            pl.BlockSpec((BM, FH), lambda i: (i, 0)),
            pl.BlockSpec((NPAD,), lambda i: (0,)),
        ],
        out_shape=[
            jax.ShapeDtypeStruct((N, FH), jnp.float32),
            jax.ShapeDtypeStruct((N, FH), jnp.float32),
            jax.ShapeDtypeStruct((NPAD,), jnp.float32),
        ],
    )(x, W1, degp)


def _k4_body(t_ref, hsl_ref, hsh_ref, dinv_ref, b1_ref, w2_ref, gs_ref):
    i = pl.program_id(0)
    dinv = dinv_ref[pl.ds(i * BM, BM)]
    b1 = b1_ref[...]
    al = (t_ref[0] + hsl_ref[...]) * dinv[:, None] + b1[None, :FH]
    ah = (t_ref[1] + hsh_ref[...]) * dinv[:, None] + b1[None, FH:]
    g = (jnp.dot(jnp.maximum(al, 0.0), w2_ref[pl.ds(0, FH), :],
                 preferred_element_type=jnp.float32)
         + jnp.dot(jnp.maximum(ah, 0.0), w2_ref[pl.ds(FH, FH), :],
                   preferred_element_type=jnp.float32))
    gs_ref[...] = g * dinv[:, None]


def _k4(t, hsl, hsh, dinv, b1, W2p):
    return pl.pallas_call(
        _k4_body,
        grid=(pl.cdiv(N, BM),),
        in_specs=[
            pl.BlockSpec((NC, BM, FH), lambda i: (0, i, 0)),
            pl.BlockSpec((BM, FH), lambda i: (i, 0)),
            pl.BlockSpec((BM, FH), lambda i: (i, 0)),
            pl.BlockSpec((NPAD,), lambda i: (0,)),
            pl.BlockSpec((H,), lambda i: (0,)),
            pl.BlockSpec((H, 16), lambda i: (0, 0)),
        ],
        out_specs=pl.BlockSpec((BM, 16), lambda i: (i, 0)),
        out_shape=jax.ShapeDtypeStruct((N, 16), jnp.float32),
    )(t, hsl, hsh, dinv, b1, W2p)


def _k6_body(t2a_ref, t2b_ref, gs_ref, dinv_ref, b2_ref, out_ref):
    i = pl.program_id(0)
    dinv = dinv_ref[pl.ds(i * BM, BM)]
    z = (t2a_ref[...] + t2b_ref[...] + gs_ref[...]) * dinv[:, None]
    z2 = z[:, :2] + b2_ref[...][None, :]
    m = jnp.max(z2, axis=1, keepdims=True)
    lse = m + jnp.log(jnp.sum(jnp.exp(z2 - m), axis=1, keepdims=True))
    out_ref[...] = z2 - lse


def _k6(t2a, t2b, gs, dinv, b2):
    return pl.pallas_call(
        _k6_body,
        grid=(pl.cdiv(N, BM),),
        in_specs=[
            pl.BlockSpec((BM, 16), lambda i: (i, 0)),
            pl.BlockSpec((BM, 16), lambda i: (i, 0)),
            pl.BlockSpec((BM, 16), lambda i: (i, 0)),
            pl.BlockSpec((NPAD,), lambda i: (0,)),
            pl.BlockSpec((2,), lambda i: (0,)),
        ],
        out_specs=pl.BlockSpec((BM, 2), lambda i: (i, 0)),
        out_shape=jax.ShapeDtypeStruct((N, 2), jnp.float32),
    )(t2a, t2b, gs, dinv, b2)


def kernel(x, edge_index, W1, b1, W2, b2):
    src = edge_index[0]
    dst = edge_index[1]
    src4 = src.reshape(NC, NS, E // (NC * NS * CH), CH)
    dst4 = dst.reshape(NC, NS, E // (NC * NS * CH), CH)

    ones_ch = jnp.ones((CH,), jnp.float32)
    zer_stripe = jnp.zeros((STRIPE,), jnp.float32)
    zer_128 = jnp.zeros((8, FH), jnp.float32)
    zer_s16 = jnp.zeros((STRIPE, 16), jnp.float32)
    W2p = jnp.zeros((H, 16), jnp.float32).at[:, :2].set(W2)

    degp = _deg(dst4, ones_ch, zer_stripe)
    hsl, hsh, dinv = _k2(x, W1, degp)
    t = _agg1(hsl, hsh, src, dst, zer_128)
    gs = _k4(t, hsl, hsh, dinv, b1, W2p)
    t2 = _agg2(gs, src4, dst4, zer_s16)
    return _k6(t2[0], t2[1], gs, dinv, b2)
